# baseline (speedup 1.0000x reference)
import functools

import jax
import jax.numpy as jnp
from jax import lax
from jax.experimental import pallas as pl
from jax.experimental.pallas import tpu as pltpu
from jax.experimental.pallas import tpu_sc as plsc

B, N, C, H, W = 2, 1024, 96, 16, 16
HW = H * W
RROWS = C * HW
NROWS = B * N

_CGRP = 24


def _pool_body(x_ref, out_ref):
    xb = x_ref[0].reshape(_CGRP, HW, N)
    out_ref[0] = jnp.mean(xb, axis=1)


def _pool(xt):
    return pl.pallas_call(
        _pool_body,
        grid=(B, C // _CGRP),
        in_specs=[pl.BlockSpec((1, _CGRP * HW, N), lambda b, i: (b, i, 0))],
        out_specs=pl.BlockSpec((1, _CGRP, N), lambda b, i: (b, i, 0)),
        out_shape=jax.ShapeDtypeStruct((B, C, N), jnp.float32),
    )(xt)


def _attn_body(pooled_ref, w_ref, b_ref, idx_ref):
    a = pooled_ref[0]
    mu = jnp.mean(a, axis=0, keepdims=True)
    var = jnp.mean((a - mu) ** 2, axis=0, keepdims=True)
    q = (a - mu) / jnp.sqrt(var + 1e-5) * w_ref[...] + b_ref[...]
    attn = lax.dot_general(
        q, q, (((0,), (0,)), ((), ())),
        preferred_element_type=jnp.float32,
    ) * (C ** -0.5)
    rows = lax.broadcasted_iota(jnp.int32, (N, N), 0)
    cols = lax.broadcasted_iota(jnp.int32, (N, N), 1)
    attn = jnp.where(rows == cols, attn - 100.0, attn)
    m = jnp.max(attn, axis=-1, keepdims=True)
    cand = jnp.where(attn == m, cols, N)
    idx_ref[0, 0] = jnp.min(cand, axis=-1)


def _attn_argmax(pooled3, ln_w, ln_b):
    return pl.pallas_call(
        _attn_body,
        grid=(B,),
        in_specs=[
            pl.BlockSpec((1, C, N), lambda i: (i, 0, 0)),
            pl.BlockSpec((C, 1), lambda i: (0, 0)),
            pl.BlockSpec((C, 1), lambda i: (0, 0)),
        ],
        out_specs=pl.BlockSpec((1, 1, N), lambda i: (i, 0, 0)),
        out_shape=jax.ShapeDtypeStruct((B, 1, N), jnp.int32),
    )(pooled3, ln_w.reshape(C, 1), ln_b.reshape(C, 1))


_INFO = plsc.get_sparse_core_info()
_NW = _INFO.num_cores * _INFO.num_subcores
_RW = RROWS // _NW
_RB = 16
_NCH = _RW // _RB

_SC_MESH = plsc.VectorSubcoreMesh(core_axis_name="c", subcore_axis_name="s")


def _permute_chunk(in_ref, idx_ref, o0_ref, o1_ref):
    @plsc.parallel_loop(0, N // 16, unroll=4)
    def jbody(j):
        for b in range(B):
            o_ref = (o0_ref, o1_ref)[b]
            cv = idx_ref[pl.ds(b * N + j * 16, 16)]
            for r in range(_RB):
                rv = jnp.full((16,), r, jnp.int32)
                vals = plsc.load_gather(in_ref, [rv, cv])
                o_ref[r, pl.ds(j * 16, 16)] = vals


@functools.partial(
    pl.kernel,
    out_type=jax.ShapeDtypeStruct((B, RROWS, N), jnp.float32),
    mesh=_SC_MESH,
    compiler_params=pltpu.CompilerParams(
        use_tc_tiling_on_sc=True, needs_layout_passes=False),
    scratch_types=[
        pltpu.VMEM((NROWS,), jnp.int32),
        pltpu.VMEM((_RB, N), jnp.float32),
        pltpu.VMEM((_RB, N), jnp.float32),
        pltpu.VMEM((_RB, N), jnp.float32),
        pltpu.VMEM((_RB, N), jnp.float32),
        pltpu.VMEM((_RB, N), jnp.float32),
        pltpu.VMEM((_RB, N), jnp.float32),
        pltpu.SemaphoreType.DMA,
        pltpu.SemaphoreType.DMA,
        pltpu.SemaphoreType.DMA,
        pltpu.SemaphoreType.DMA,
    ],
)
def _sc_gather(xt_hbm, idx_hbm, out_hbm, idx_v, in_a, in_b,
               o0_a, o1_a, o0_b, o1_b, sin_a, sin_b, sout_a, sout_b):
    wid = lax.axis_index("s") * _INFO.num_cores + lax.axis_index("c")
    base = wid * _RW
    pltpu.sync_copy(idx_hbm, idx_v)
    pltpu.async_copy(xt_hbm.at[0, pl.ds(base, _RB)], in_a, sin_a)

    def pair(p, carry):
        k0 = 2 * p
        pltpu.async_copy(
            xt_hbm.at[0, pl.ds(base + (k0 + 1) * _RB, _RB)], in_b, sin_b)
        pltpu.make_async_copy(xt_hbm.at[0, pl.ds(0, _RB)], in_a, sin_a).wait()

        @pl.when(k0 >= 2)
        def _():
            pltpu.make_async_copy(o0_a, out_hbm.at[0, pl.ds(0, _RB)], sout_a).wait()
            pltpu.make_async_copy(o1_a, out_hbm.at[1, pl.ds(0, _RB)], sout_a).wait()

        _permute_chunk(in_a, idx_v, o0_a, o1_a)
        row0 = base + k0 * _RB
        pltpu.async_copy(o0_a, out_hbm.at[0, pl.ds(row0, _RB)], sout_a)
        pltpu.async_copy(o1_a, out_hbm.at[1, pl.ds(row0, _RB)], sout_a)

        @pl.when(k0 + 2 < _NCH)
        def _():
            pltpu.async_copy(
                xt_hbm.at[0, pl.ds(base + (k0 + 2) * _RB, _RB)], in_a, sin_a)

        pltpu.make_async_copy(xt_hbm.at[0, pl.ds(0, _RB)], in_b, sin_b).wait()

        @pl.when(k0 >= 2)
        def _():
            pltpu.make_async_copy(o0_b, out_hbm.at[0, pl.ds(0, _RB)], sout_b).wait()
            pltpu.make_async_copy(o1_b, out_hbm.at[1, pl.ds(0, _RB)], sout_b).wait()

        _permute_chunk(in_b, idx_v, o0_b, o1_b)
        row1 = base + (k0 + 1) * _RB
        pltpu.async_copy(o0_b, out_hbm.at[0, pl.ds(row1, _RB)], sout_b)
        pltpu.async_copy(o1_b, out_hbm.at[1, pl.ds(row1, _RB)], sout_b)
        return carry

    lax.fori_loop(0, _NCH // 2, pair, 0)
    pltpu.make_async_copy(o0_a, out_hbm.at[0, pl.ds(0, _RB)], sout_a).wait()
    pltpu.make_async_copy(o1_a, out_hbm.at[1, pl.ds(0, _RB)], sout_a).wait()
    pltpu.make_async_copy(o0_b, out_hbm.at[0, pl.ds(0, _RB)], sout_b).wait()
    pltpu.make_async_copy(o1_b, out_hbm.at[1, pl.ds(0, _RB)], sout_b).wait()


def kernel(x, ln_w, ln_b):
    xt = x.transpose(0, 2, 3, 4, 1).reshape(B, RROWS, N)
    pooled = _pool(xt)
    idx = _attn_argmax(pooled, ln_w, ln_b)
    yt = _sc_gather(xt, idx.reshape(NROWS))
    return yt.reshape(B, C, H, W, N).transpose(0, 4, 1, 2, 3)


# speedup vs baseline: 1.0107x; 1.0107x over previous
import functools

import jax
import jax.numpy as jnp
from jax import lax
from jax.experimental import pallas as pl
from jax.experimental.pallas import tpu as pltpu
from jax.experimental.pallas import tpu_sc as plsc

B, N, C, H, W = 2, 1024, 96, 16, 16
HW = H * W
RROWS = C * HW
NROWS = B * N

_CGRP = 16


def _pool_body(x_ref, out_ref):
    xb = x_ref[0].reshape(_CGRP, HW, N)
    out_ref[0] = jnp.mean(xb, axis=1)


def _pool(xt):
    return pl.pallas_call(
        _pool_body,
        grid=(B, C // _CGRP),
        in_specs=[pl.BlockSpec((1, _CGRP * HW, N), lambda b, i: (b, i, 0))],
        out_specs=pl.BlockSpec((1, _CGRP, N), lambda b, i: (b, i, 0)),
        out_shape=jax.ShapeDtypeStruct((B, C, N), jnp.float32),
    )(xt)


def _attn_body(pooled_ref, w_ref, b_ref, idx_ref):
    a = pooled_ref[0]
    mu = jnp.mean(a, axis=0, keepdims=True)
    var = jnp.mean((a - mu) ** 2, axis=0, keepdims=True)
    q = (a - mu) / jnp.sqrt(var + 1e-5) * w_ref[...] + b_ref[...]
    attn = lax.dot_general(
        q, q, (((0,), (0,)), ((), ())),
        preferred_element_type=jnp.float32,
    ) * (C ** -0.5)
    rows = lax.broadcasted_iota(jnp.int32, (N, N), 0)
    cols = lax.broadcasted_iota(jnp.int32, (N, N), 1)
    attn = jnp.where(rows == cols, attn - 100.0, attn)
    m = jnp.max(attn, axis=-1, keepdims=True)
    cand = jnp.where(attn == m, cols, N)
    idx_ref[0, 0] = jnp.min(cand, axis=-1)


def _attn_argmax(pooled3, ln_w, ln_b):
    return pl.pallas_call(
        _attn_body,
        grid=(B,),
        in_specs=[
            pl.BlockSpec((1, C, N), lambda i: (i, 0, 0)),
            pl.BlockSpec((C, 1), lambda i: (0, 0)),
            pl.BlockSpec((C, 1), lambda i: (0, 0)),
        ],
        out_specs=pl.BlockSpec((1, 1, N), lambda i: (i, 0, 0)),
        out_shape=jax.ShapeDtypeStruct((B, 1, N), jnp.int32),
    )(pooled3, ln_w.reshape(C, 1), ln_b.reshape(C, 1))


_INFO = plsc.get_sparse_core_info()
_NW = _INFO.num_cores * _INFO.num_subcores
_RW = RROWS // _NW
_RB = 16
_NCH = _RW // _RB

_SC_MESH = plsc.VectorSubcoreMesh(core_axis_name="c", subcore_axis_name="s")


def _permute_chunk(in_ref, idx_ref, o0_ref, o1_ref):
    @plsc.parallel_loop(0, N // 16, unroll=4)
    def jbody(j):
        for b in range(B):
            o_ref = (o0_ref, o1_ref)[b]
            cv = idx_ref[pl.ds(b * N + j * 16, 16)]
            for r in range(_RB):
                rv = jnp.full((16,), r, jnp.int32)
                vals = plsc.load_gather(in_ref, [rv, cv])
                o_ref[r, pl.ds(j * 16, 16)] = vals


@functools.partial(
    pl.kernel,
    out_type=jax.ShapeDtypeStruct((B, RROWS, N), jnp.float32),
    mesh=_SC_MESH,
    compiler_params=pltpu.CompilerParams(
        use_tc_tiling_on_sc=True, needs_layout_passes=False),
    scratch_types=[
        pltpu.VMEM((NROWS,), jnp.int32),
        pltpu.VMEM((_RB, N), jnp.float32),
        pltpu.VMEM((_RB, N), jnp.float32),
        pltpu.VMEM((_RB, N), jnp.float32),
        pltpu.VMEM((_RB, N), jnp.float32),
        pltpu.VMEM((_RB, N), jnp.float32),
        pltpu.VMEM((_RB, N), jnp.float32),
        pltpu.SemaphoreType.DMA,
        pltpu.SemaphoreType.DMA,
        pltpu.SemaphoreType.DMA,
        pltpu.SemaphoreType.DMA,
    ],
)
def _sc_gather(xt_hbm, idx_hbm, out_hbm, idx_v, in_a, in_b,
               o0_a, o1_a, o0_b, o1_b, sin_a, sin_b, sout_a, sout_b):
    wid = lax.axis_index("s") * _INFO.num_cores + lax.axis_index("c")
    base = wid * _RW
    pltpu.sync_copy(idx_hbm, idx_v)
    pltpu.async_copy(xt_hbm.at[0, pl.ds(base, _RB)], in_a, sin_a)

    def pair(p, carry):
        k0 = 2 * p
        pltpu.async_copy(
            xt_hbm.at[0, pl.ds(base + (k0 + 1) * _RB, _RB)], in_b, sin_b)
        pltpu.make_async_copy(xt_hbm.at[0, pl.ds(0, _RB)], in_a, sin_a).wait()

        @pl.when(k0 >= 2)
        def _():
            pltpu.make_async_copy(o0_a, out_hbm.at[0, pl.ds(0, _RB)], sout_a).wait()
            pltpu.make_async_copy(o1_a, out_hbm.at[1, pl.ds(0, _RB)], sout_a).wait()

        _permute_chunk(in_a, idx_v, o0_a, o1_a)
        row0 = base + k0 * _RB
        pltpu.async_copy(o0_a, out_hbm.at[0, pl.ds(row0, _RB)], sout_a)
        pltpu.async_copy(o1_a, out_hbm.at[1, pl.ds(row0, _RB)], sout_a)

        @pl.when(k0 + 2 < _NCH)
        def _():
            pltpu.async_copy(
                xt_hbm.at[0, pl.ds(base + (k0 + 2) * _RB, _RB)], in_a, sin_a)

        pltpu.make_async_copy(xt_hbm.at[0, pl.ds(0, _RB)], in_b, sin_b).wait()

        @pl.when(k0 >= 2)
        def _():
            pltpu.make_async_copy(o0_b, out_hbm.at[0, pl.ds(0, _RB)], sout_b).wait()
            pltpu.make_async_copy(o1_b, out_hbm.at[1, pl.ds(0, _RB)], sout_b).wait()

        _permute_chunk(in_b, idx_v, o0_b, o1_b)
        row1 = base + (k0 + 1) * _RB
        pltpu.async_copy(o0_b, out_hbm.at[0, pl.ds(row1, _RB)], sout_b)
        pltpu.async_copy(o1_b, out_hbm.at[1, pl.ds(row1, _RB)], sout_b)
        return carry

    lax.fori_loop(0, _NCH // 2, pair, 0)
    pltpu.make_async_copy(o0_a, out_hbm.at[0, pl.ds(0, _RB)], sout_a).wait()
    pltpu.make_async_copy(o1_a, out_hbm.at[1, pl.ds(0, _RB)], sout_a).wait()
    pltpu.make_async_copy(o0_b, out_hbm.at[0, pl.ds(0, _RB)], sout_b).wait()
    pltpu.make_async_copy(o1_b, out_hbm.at[1, pl.ds(0, _RB)], sout_b).wait()


def kernel(x, ln_w, ln_b):
    xt = x.transpose(0, 2, 3, 4, 1).reshape(B, RROWS, N)
    pooled = _pool(xt)
    idx = _attn_argmax(pooled, ln_w, ln_b)
    yt = _sc_gather(xt, idx.reshape(NROWS))
    return yt.reshape(B, C, H, W, N).transpose(0, 4, 1, 2, 3)


# speedup vs baseline: 1.1396x; 1.1276x over previous
import functools

import jax
import jax.numpy as jnp
from jax import lax
from jax.experimental import pallas as pl
from jax.experimental.pallas import tpu as pltpu
from jax.experimental.pallas import tpu_sc as plsc

B, N, C, H, W = 2, 1024, 96, 16, 16
HW = H * W
RROWS = C * HW
NROWS = B * N

_CGRP = 16


def _pool_body(x_ref, out_ref):
    xb = x_ref[0].reshape(_CGRP, HW, N)
    out_ref[0] = jnp.mean(xb, axis=1)


def _pool(xt):
    return pl.pallas_call(
        _pool_body,
        grid=(B, C // _CGRP),
        in_specs=[pl.BlockSpec((1, _CGRP * HW, N), lambda b, i: (b, i, 0))],
        out_specs=pl.BlockSpec((1, _CGRP, N), lambda b, i: (b, i, 0)),
        out_shape=jax.ShapeDtypeStruct((B, C, N), jnp.float32),
    )(xt)


def _attn_body(pooled_ref, w_ref, b_ref, idx_ref):
    a = pooled_ref[0]
    mu = jnp.mean(a, axis=0, keepdims=True)
    var = jnp.mean((a - mu) ** 2, axis=0, keepdims=True)
    q = (a - mu) / jnp.sqrt(var + 1e-5) * w_ref[...] + b_ref[...]
    attn = lax.dot_general(
        q, q, (((0,), (0,)), ((), ())),
        preferred_element_type=jnp.float32,
    ) * (C ** -0.5)
    rows = lax.broadcasted_iota(jnp.int32, (N, N), 0)
    cols = lax.broadcasted_iota(jnp.int32, (N, N), 1)
    attn = jnp.where(rows == cols, attn - 100.0, attn)
    m = jnp.max(attn, axis=-1, keepdims=True)
    cand = jnp.where(attn == m, cols, N)
    idx_ref[0, 0] = jnp.min(cand, axis=-1)


def _attn_argmax(pooled3, ln_w, ln_b):
    return pl.pallas_call(
        _attn_body,
        grid=(B,),
        in_specs=[
            pl.BlockSpec((1, C, N), lambda i: (i, 0, 0)),
            pl.BlockSpec((C, 1), lambda i: (0, 0)),
            pl.BlockSpec((C, 1), lambda i: (0, 0)),
        ],
        out_specs=pl.BlockSpec((1, 1, N), lambda i: (i, 0, 0)),
        out_shape=jax.ShapeDtypeStruct((B, 1, N), jnp.int32),
    )(pooled3, ln_w.reshape(C, 1), ln_b.reshape(C, 1))


_INFO = plsc.get_sparse_core_info()
_NW = _INFO.num_cores * _INFO.num_subcores
_RW = RROWS // _NW
_RB = 16
_NCH = _RW // _RB

_SC_MESH = plsc.VectorSubcoreMesh(core_axis_name="c", subcore_axis_name="s")


def _permute_chunk(in_ref, idx_ref, o0_ref, o1_ref):
    @plsc.parallel_loop(0, N // 16, unroll=2)
    def jbody(j):
        for b in range(B):
            o_ref = (o0_ref, o1_ref)[b]
            cv = idx_ref[pl.ds(b * N + j * 16, 16)]
            for r in range(_RB):
                rv = jnp.full((16,), r, jnp.int32)
                vals = plsc.load_gather(in_ref, [rv, cv])
                o_ref[r, pl.ds(j * 16, 16)] = vals


@functools.partial(
    pl.kernel,
    out_type=jax.ShapeDtypeStruct((B, RROWS, N), jnp.float32),
    mesh=_SC_MESH,
    compiler_params=pltpu.CompilerParams(
        use_tc_tiling_on_sc=True, needs_layout_passes=False),
    scratch_types=[
        pltpu.VMEM((NROWS,), jnp.int32),
        pltpu.VMEM((_RB, N), jnp.float32),
        pltpu.VMEM((_RB, N), jnp.float32),
        pltpu.VMEM((_RB, N), jnp.float32),
        pltpu.VMEM((_RB, N), jnp.float32),
        pltpu.VMEM((_RB, N), jnp.float32),
        pltpu.VMEM((_RB, N), jnp.float32),
        pltpu.SemaphoreType.DMA,
        pltpu.SemaphoreType.DMA,
        pltpu.SemaphoreType.DMA,
        pltpu.SemaphoreType.DMA,
    ],
)
def _sc_gather(xt_hbm, idx_hbm, out_hbm, idx_v, in_a, in_b,
               o0_a, o1_a, o0_b, o1_b, sin_a, sin_b, sout_a, sout_b):
    wid = lax.axis_index("s") * _INFO.num_cores + lax.axis_index("c")
    base = wid * _RW
    pltpu.sync_copy(idx_hbm, idx_v)
    pltpu.async_copy(xt_hbm.at[0, pl.ds(base, _RB)], in_a, sin_a)

    def pair(p, carry):
        k0 = 2 * p
        pltpu.async_copy(
            xt_hbm.at[0, pl.ds(base + (k0 + 1) * _RB, _RB)], in_b, sin_b)
        pltpu.make_async_copy(xt_hbm.at[0, pl.ds(0, _RB)], in_a, sin_a).wait()

        @pl.when(k0 >= 2)
        def _():
            pltpu.make_async_copy(o0_a, out_hbm.at[0, pl.ds(0, _RB)], sout_a).wait()
            pltpu.make_async_copy(o1_a, out_hbm.at[1, pl.ds(0, _RB)], sout_a).wait()

        _permute_chunk(in_a, idx_v, o0_a, o1_a)
        row0 = base + k0 * _RB
        pltpu.async_copy(o0_a, out_hbm.at[0, pl.ds(row0, _RB)], sout_a)
        pltpu.async_copy(o1_a, out_hbm.at[1, pl.ds(row0, _RB)], sout_a)

        @pl.when(k0 + 2 < _NCH)
        def _():
            pltpu.async_copy(
                xt_hbm.at[0, pl.ds(base + (k0 + 2) * _RB, _RB)], in_a, sin_a)

        pltpu.make_async_copy(xt_hbm.at[0, pl.ds(0, _RB)], in_b, sin_b).wait()

        @pl.when(k0 >= 2)
        def _():
            pltpu.make_async_copy(o0_b, out_hbm.at[0, pl.ds(0, _RB)], sout_b).wait()
            pltpu.make_async_copy(o1_b, out_hbm.at[1, pl.ds(0, _RB)], sout_b).wait()

        _permute_chunk(in_b, idx_v, o0_b, o1_b)
        row1 = base + (k0 + 1) * _RB
        pltpu.async_copy(o0_b, out_hbm.at[0, pl.ds(row1, _RB)], sout_b)
        pltpu.async_copy(o1_b, out_hbm.at[1, pl.ds(row1, _RB)], sout_b)
        return carry

    lax.fori_loop(0, _NCH // 2, pair, 0)
    pltpu.make_async_copy(o0_a, out_hbm.at[0, pl.ds(0, _RB)], sout_a).wait()
    pltpu.make_async_copy(o1_a, out_hbm.at[1, pl.ds(0, _RB)], sout_a).wait()
    pltpu.make_async_copy(o0_b, out_hbm.at[0, pl.ds(0, _RB)], sout_b).wait()
    pltpu.make_async_copy(o1_b, out_hbm.at[1, pl.ds(0, _RB)], sout_b).wait()


def kernel(x, ln_w, ln_b):
    xt = x.transpose(0, 2, 3, 4, 1).reshape(B, RROWS, N)
    pooled = _pool(xt)
    idx = _attn_argmax(pooled, ln_w, ln_b)
    yt = _sc_gather(xt, idx.reshape(NROWS))
    return yt.reshape(B, C, H, W, N).transpose(0, 4, 1, 2, 3)
